# baseline (speedup 1.0000x reference)
import functools
import math

import jax
import jax.numpy as jnp
import numpy as np
from jax import lax
from jax.experimental import pallas as pl
from jax.experimental.pallas import tpu as pltpu
from jax.experimental.pallas import tpu_sc as plsc

VOCAB = 100000
MAX_TOKENS = 2048
D_MODEL = 768
SCALE = math.sqrt(float(D_MODEL))

NUM_CORES = 2
NUM_SUBCORES = 16
NUM_WORKERS = NUM_CORES * NUM_SUBCORES

B, T = 4, MAX_TOKENS
TOTAL = B * T
PER_W = TOTAL // NUM_WORKERS
CHUNK = 64
NCHUNK = PER_W // CHUNK
LANES = 16
VECS_PER_ROW = D_MODEL // LANES


def _pe_table() -> np.ndarray:
    positions = np.arange(MAX_TOKENS)[:, np.newaxis]
    d_half = D_MODEL // 2
    d_scales = (1.0 / 10000 ** (np.arange(d_half) / d_half))[np.newaxis, :]
    pe = np.empty((MAX_TOKENS, D_MODEL), dtype=np.float32)
    pe[:, 0::2] = np.sin(positions * d_scales)
    pe[:, 1::2] = np.cos(positions * d_scales)
    return pe


_PE = _pe_table()

_mesh = plsc.VectorSubcoreMesh(
    core_axis_name="c",
    subcore_axis_name="s",
    num_cores=NUM_CORES,
    num_subcores=NUM_SUBCORES,
)


@functools.partial(
    pl.kernel,
    out_type=jax.ShapeDtypeStruct((TOTAL, D_MODEL), jnp.float32),
    mesh=_mesh,
    scratch_types=[
        pltpu.VMEM((NCHUNK, CHUNK), jnp.int32),
        pltpu.VMEM((CHUNK, D_MODEL), jnp.float32),
        pltpu.VMEM((CHUNK, D_MODEL), jnp.float32),
        pltpu.SemaphoreType.DMA,
    ],
)
def _emb_kernel(xf_hbm, table_hbm, pe_hbm, out_hbm, idx_v, rows, pe_v, sem):
    wid = lax.axis_index("s") * NUM_CORES + lax.axis_index("c")
    base = wid * PER_W
    pos0 = lax.rem(base, MAX_TOKENS)
    pltpu.sync_copy(xf_hbm.at[wid], idx_v)
    for c in range(NCHUNK):
        gather = pltpu.async_copy(table_hbm.at[idx_v.at[c]], rows, sem)
        pltpu.sync_copy(pe_hbm.at[pl.ds(pos0 + c * CHUNK, CHUNK)], pe_v)
        gather.wait()

        def _fma_row(r, carry):
            for j in range(VECS_PER_ROW):
                sl = pl.ds(j * LANES, LANES)
                rows[r, sl] = rows[r, sl] * SCALE + pe_v[r, sl]
            return carry

        lax.fori_loop(0, CHUNK, _fma_row, 0)
        pltpu.sync_copy(rows, out_hbm.at[pl.ds(base + c * CHUNK, CHUNK)])


def kernel(x, table):
    xf = x.reshape(NUM_WORKERS, NCHUNK, CHUNK)
    out = _emb_kernel(xf, table, _PE)
    return out.reshape(B, T, D_MODEL)


# speedup vs baseline: 1.3258x; 1.3258x over previous
import functools
import math

import jax
import jax.numpy as jnp
import numpy as np
from jax import lax
from jax.experimental import pallas as pl
from jax.experimental.pallas import tpu as pltpu
from jax.experimental.pallas import tpu_sc as plsc

VOCAB = 100000
MAX_TOKENS = 2048
D_MODEL = 768
SCALE = math.sqrt(float(D_MODEL))

NUM_CORES = 2
NUM_SUBCORES = 16
NUM_WORKERS = NUM_CORES * NUM_SUBCORES

B, T = 4, MAX_TOKENS
TOTAL = B * T
POS_PER_W = T // NUM_WORKERS
NCHUNK = 8
POS_PER_CHUNK = POS_PER_W // NCHUNK
ROWS_PER_CHUNK = POS_PER_CHUNK * B
LANES = 16
VECS_PER_ROW = D_MODEL // LANES


def _pe_table() -> np.ndarray:
    positions = np.arange(MAX_TOKENS)[:, np.newaxis]
    d_half = D_MODEL // 2
    d_scales = (1.0 / 10000 ** (np.arange(d_half) / d_half))[np.newaxis, :]
    pe = np.empty((MAX_TOKENS, D_MODEL), dtype=np.float32)
    pe[:, 0::2] = np.sin(positions * d_scales)
    pe[:, 1::2] = np.cos(positions * d_scales)
    return pe


_PE = _pe_table()

_mesh = plsc.VectorSubcoreMesh(
    core_axis_name="c",
    subcore_axis_name="s",
    num_cores=NUM_CORES,
    num_subcores=NUM_SUBCORES,
)


@functools.partial(
    pl.kernel,
    out_type=jax.ShapeDtypeStruct((TOTAL, D_MODEL), jnp.float32),
    mesh=_mesh,
    scratch_types=[
        pltpu.VMEM((NCHUNK, ROWS_PER_CHUNK), jnp.int32),
        pltpu.VMEM((POS_PER_W, D_MODEL), jnp.float32),
        pltpu.VMEM((ROWS_PER_CHUNK, D_MODEL), jnp.float32),
        pltpu.VMEM((ROWS_PER_CHUNK, D_MODEL), jnp.float32),
        pltpu.SemaphoreType.DMA,
        pltpu.SemaphoreType.DMA,
        pltpu.SemaphoreType.DMA,
        pltpu.SemaphoreType.DMA,
    ],
)
def _emb_kernel(xf_hbm, table_hbm, pe_hbm, out_hbm,
                idx_v, pe_v, rows0, rows1, g0, g1, w0, w1):
    wid = lax.axis_index("s") * NUM_CORES + lax.axis_index("c")
    pos0 = wid * POS_PER_W
    rbufs = (rows0, rows1)
    gsems = (g0, g1)
    wsems = (w0, w1)

    pltpu.sync_copy(xf_hbm.at[wid], idx_v)
    pltpu.async_copy(table_hbm.at[idx_v.at[0]], rows0, g0)
    pltpu.sync_copy(pe_hbm.at[pl.ds(pos0, POS_PER_W)], pe_v)

    wb_pending = [None, None]
    for c in range(NCHUNK):
        sel = c % 2
        nxt = (c + 1) % 2
        if c + 1 < NCHUNK:
            if wb_pending[nxt] is not None:
                for d in wb_pending[nxt]:
                    d.wait()
                wb_pending[nxt] = None
            pltpu.async_copy(table_hbm.at[idx_v.at[c + 1]], rbufs[nxt],
                             gsems[nxt])
        pltpu.make_async_copy(table_hbm.at[idx_v.at[c]], rbufs[sel],
                              gsems[sel]).wait()

        rbuf = rbufs[sel]
        pbase = c * POS_PER_CHUNK

        def _fma(r, carry):
            for j in range(VECS_PER_ROW):
                sl = pl.ds(j * LANES, LANES)
                pe_vec = pe_v[pbase + r, sl]
                for b in range(B):
                    row = b * POS_PER_CHUNK + r
                    rbuf[row, sl] = rbuf[row, sl] * SCALE + pe_vec
            return carry

        lax.fori_loop(0, POS_PER_CHUNK, _fma, 0)

        descs = []
        for b in range(B):
            dst = out_hbm.at[
                pl.ds(b * T + pos0 + c * POS_PER_CHUNK, POS_PER_CHUNK)]
            src = rbuf.at[pl.ds(b * POS_PER_CHUNK, POS_PER_CHUNK)]
            descs.append(pltpu.async_copy(src, dst, wsems[sel]))
        wb_pending[sel] = descs

    for pending in wb_pending:
        if pending is not None:
            for d in pending:
                d.wait()


def kernel(x, table):
    xf = (x.reshape(B, NUM_WORKERS, NCHUNK, POS_PER_CHUNK)
          .transpose(1, 2, 0, 3)
          .reshape(NUM_WORKERS, NCHUNK, ROWS_PER_CHUNK))
    out = _emb_kernel(xf, table, _PE)
    return out.reshape(B, T, D_MODEL)


# speedup vs baseline: 1.3530x; 1.0206x over previous
import functools
import math

import jax
import jax.numpy as jnp
import numpy as np
from jax import lax
from jax.experimental import pallas as pl
from jax.experimental.pallas import tpu as pltpu
from jax.experimental.pallas import tpu_sc as plsc

VOCAB = 100000
MAX_TOKENS = 2048
D_MODEL = 768
SCALE = math.sqrt(float(D_MODEL))

NUM_CORES = 2
NUM_SUBCORES = 16
NUM_WORKERS = NUM_CORES * NUM_SUBCORES

B, T = 4, MAX_TOKENS
TOTAL = B * T
POS_PER_W = T // NUM_WORKERS
NCHUNK = 8
POS_PER_CHUNK = POS_PER_W // NCHUNK
ROWS_PER_CHUNK = POS_PER_CHUNK * B
LANES = 16
VECS_PER_ROW = D_MODEL // LANES


def _pe_table() -> np.ndarray:
    positions = np.arange(MAX_TOKENS)[:, np.newaxis]
    d_half = D_MODEL // 2
    d_scales = (1.0 / 10000 ** (np.arange(d_half) / d_half))[np.newaxis, :]
    pe = np.empty((MAX_TOKENS, D_MODEL), dtype=np.float32)
    pe[:, 0::2] = np.sin(positions * d_scales)
    pe[:, 1::2] = np.cos(positions * d_scales)
    return pe


_PE = _pe_table()

_mesh = plsc.VectorSubcoreMesh(
    core_axis_name="c",
    subcore_axis_name="s",
    num_cores=NUM_CORES,
    num_subcores=NUM_SUBCORES,
)


NBUF = 3


@functools.partial(
    pl.kernel,
    out_type=jax.ShapeDtypeStruct((TOTAL, D_MODEL), jnp.float32),
    mesh=_mesh,
    scratch_types=[
        pltpu.VMEM((NCHUNK, ROWS_PER_CHUNK), jnp.int32),
        pltpu.VMEM((POS_PER_W, D_MODEL), jnp.float32),
    ]
    + [pltpu.VMEM((ROWS_PER_CHUNK, D_MODEL), jnp.float32)] * NBUF
    + [pltpu.SemaphoreType.DMA] * (2 * NBUF),
)
def _emb_kernel(xf_hbm, table_hbm, pe_hbm, out_hbm, idx_v, pe_v, *bufs_sems):
    rbufs = bufs_sems[:NBUF]
    gsems = bufs_sems[NBUF:2 * NBUF]
    wsems = bufs_sems[2 * NBUF:]
    wid = lax.axis_index("s") * NUM_CORES + lax.axis_index("c")
    pos0 = wid * POS_PER_W

    pltpu.sync_copy(xf_hbm.at[wid], idx_v)
    for c in range(NBUF - 1):
        pltpu.async_copy(table_hbm.at[idx_v.at[c]], rbufs[c], gsems[c])
    pltpu.sync_copy(pe_hbm.at[pl.ds(pos0, POS_PER_W)], pe_v)

    wb_pending = [None] * NBUF
    for c in range(NCHUNK):
        sel = c % NBUF
        if c + NBUF - 1 < NCHUNK:
            ahead = (c + NBUF - 1) % NBUF
            if wb_pending[ahead] is not None:
                for d in wb_pending[ahead]:
                    d.wait()
                wb_pending[ahead] = None
            pltpu.async_copy(table_hbm.at[idx_v.at[c + NBUF - 1]],
                             rbufs[ahead], gsems[ahead])
        pltpu.make_async_copy(table_hbm.at[idx_v.at[c]], rbufs[sel],
                              gsems[sel]).wait()

        rbuf = rbufs[sel]
        pbase = c * POS_PER_CHUNK

        def _fma(r, carry):
            for j in range(VECS_PER_ROW):
                sl = pl.ds(j * LANES, LANES)
                pe_vec = pe_v[pbase + r, sl]
                for b in range(B):
                    row = b * POS_PER_CHUNK + r
                    rbuf[row, sl] = rbuf[row, sl] * SCALE + pe_vec
            return carry

        lax.fori_loop(0, POS_PER_CHUNK, _fma, 0)

        descs = []
        for b in range(B):
            dst = out_hbm.at[
                pl.ds(b * T + pos0 + c * POS_PER_CHUNK, POS_PER_CHUNK)]
            src = rbuf.at[pl.ds(b * POS_PER_CHUNK, POS_PER_CHUNK)]
            descs.append(pltpu.async_copy(src, dst, wsems[sel]))
        wb_pending[sel] = descs

    for pending in wb_pending:
        if pending is not None:
            for d in pending:
                d.wait()


def kernel(x, table):
    xf = (x.reshape(B, NUM_WORKERS, NCHUNK, POS_PER_CHUNK)
          .transpose(1, 2, 0, 3)
          .reshape(NUM_WORKERS, NCHUNK, ROWS_PER_CHUNK))
    out = _emb_kernel(xf, table, _PE)
    return out.reshape(B, T, D_MODEL)
